# baseline (speedup 1.0000x reference)
import functools

import jax
import jax.numpy as jnp
from jax import lax
from jax.experimental import pallas as pl
from jax.experimental.pallas import tpu as pltpu

ROWS = 64
VOCAB = 1_000_000
VBLK = 4096
NBLK = (VOCAB + VBLK - 1) // VBLK


def _argmax_body(x_ref, tok_ref, vmax_ref, vidx_ref):
    i = pl.program_id(0)
    x = x_ref[...]
    col = lax.broadcasted_iota(jnp.int32, (ROWS, VBLK), 1) + i * VBLK
    valid = col < VOCAB
    xm = jnp.where(valid, x, -jnp.inf)
    bmax = jnp.max(xm, axis=1, keepdims=True)
    bidx = jnp.min(
        jnp.where(xm == bmax, col, jnp.int32(2**31 - 1)), axis=1, keepdims=True
    )

    @pl.when(i == 0)
    def _init():
        vmax_ref[...] = jnp.full((ROWS, 1), -jnp.inf, jnp.float32)
        vidx_ref[...] = jnp.zeros((ROWS, 1), jnp.int32)

    cur_max = vmax_ref[...]
    cur_idx = vidx_ref[...]
    upd = bmax > cur_max
    vmax_ref[...] = jnp.where(upd, bmax, cur_max)
    vidx_ref[...] = jnp.where(upd, bidx, cur_idx)

    @pl.when(i == pl.num_programs(0) - 1)
    def _fin():
        tok_ref[...] = vidx_ref[...]


def _onehot_body(tok_ref, out_ref):
    i = pl.program_id(0)
    col = lax.broadcasted_iota(jnp.int32, (ROWS, VBLK), 1) + i * VBLK
    tok = tok_ref[...]
    out_ref[...] = jnp.where(col == tok, 1.0, 0.0).astype(jnp.float32)


@functools.partial(jax.jit, static_argnames=("interpret",))
def _run(logits, interpret=False):
    tok2 = pl.pallas_call(
        _argmax_body,
        grid=(NBLK,),
        in_specs=[pl.BlockSpec((ROWS, VBLK), lambda i: (0, i))],
        out_specs=pl.BlockSpec((ROWS, 1), lambda i: (0, 0)),
        out_shape=jax.ShapeDtypeStruct((ROWS, 1), jnp.int32),
        scratch_shapes=[
            pltpu.VMEM((ROWS, 1), jnp.float32),
            pltpu.VMEM((ROWS, 1), jnp.int32),
        ],
        compiler_params=pltpu.CompilerParams(
            dimension_semantics=("arbitrary",)
        ),
        interpret=interpret,
    )(logits)
    probs = pl.pallas_call(
        _onehot_body,
        grid=(NBLK,),
        in_specs=[pl.BlockSpec((ROWS, 1), lambda i: (0, 0))],
        out_specs=pl.BlockSpec((ROWS, VBLK), lambda i: (0, i)),
        out_shape=jax.ShapeDtypeStruct((ROWS, VOCAB), jnp.float32),
        compiler_params=pltpu.CompilerParams(
            dimension_semantics=("arbitrary",)
        ),
        interpret=interpret,
    )(tok2)
    return tok2.reshape(ROWS), probs


def kernel(logits, eos_token_ids):
    tokens, probs = _run(logits)
    return tokens, probs


# speedup vs baseline: 1.4390x; 1.4390x over previous
import functools

import jax
import jax.numpy as jnp
from jax import lax
from jax.experimental import pallas as pl
from jax.experimental.pallas import tpu as pltpu

ROWS = 64
VOCAB = 1_000_000
VBLK = 8192
NBLK = (VOCAB + VBLK - 1) // VBLK


def _argmax_body(x_ref, tok_ref, vmax_ref, vidx_ref):
    i = pl.program_id(0)
    nb = pl.num_programs(0)
    x = x_ref[...]

    @pl.when(i == 0)
    def _init():
        vmax_ref[...] = jnp.full((ROWS, 1), -jnp.inf, jnp.float32)
        vidx_ref[...] = jnp.zeros((ROWS, 1), jnp.int32)

    bmax = jnp.max(x, axis=1, keepdims=True)

    @pl.when(i == nb - 1)
    def _tail():
        col = lax.broadcasted_iota(jnp.int32, (ROWS, VBLK), 1) + i * VBLK
        xm = jnp.where(col < VOCAB, x, -jnp.inf)
        vmax_tail = jnp.max(xm, axis=1, keepdims=True)
        upd = vmax_tail > vmax_ref[...]
        bidx = jnp.min(
            jnp.where(xm == vmax_tail, col, jnp.int32(2**31 - 1)),
            axis=1, keepdims=True,
        )
        vidx_ref[...] = jnp.where(upd, bidx, vidx_ref[...])
        vmax_ref[...] = jnp.where(upd, vmax_tail, vmax_ref[...])
        tok_ref[...] = vidx_ref[...]

    @pl.when(jnp.logical_and(i < nb - 1, jnp.any(bmax > vmax_ref[...])))
    def _update():
        upd = bmax > vmax_ref[...]
        col = lax.broadcasted_iota(jnp.int32, (ROWS, VBLK), 1) + i * VBLK
        bidx = jnp.min(
            jnp.where(x == bmax, col, jnp.int32(2**31 - 1)),
            axis=1, keepdims=True,
        )
        vidx_ref[...] = jnp.where(upd, bidx, vidx_ref[...])
        vmax_ref[...] = jnp.where(upd, bmax, vmax_ref[...])


def _onehot_body(tok_ref, out_ref):
    i = pl.program_id(0)
    col = lax.broadcasted_iota(jnp.int32, (ROWS, VBLK), 1) + i * VBLK
    tok = tok_ref[...]
    out_ref[...] = jnp.where(col == tok, 1.0, 0.0).astype(jnp.float32)


@functools.partial(jax.jit, static_argnames=("interpret",))
def _run(logits, interpret=False):
    tok2 = pl.pallas_call(
        _argmax_body,
        grid=(NBLK,),
        in_specs=[pl.BlockSpec((ROWS, VBLK), lambda i: (0, i))],
        out_specs=pl.BlockSpec((ROWS, 1), lambda i: (0, 0)),
        out_shape=jax.ShapeDtypeStruct((ROWS, 1), jnp.int32),
        scratch_shapes=[
            pltpu.VMEM((ROWS, 1), jnp.float32),
            pltpu.VMEM((ROWS, 1), jnp.int32),
        ],
        compiler_params=pltpu.CompilerParams(
            dimension_semantics=("arbitrary",)
        ),
        interpret=interpret,
    )(logits)
    probs = pl.pallas_call(
        _onehot_body,
        grid=(NBLK,),
        in_specs=[pl.BlockSpec((ROWS, 1), lambda i: (0, 0))],
        out_specs=pl.BlockSpec((ROWS, VBLK), lambda i: (0, i)),
        out_shape=jax.ShapeDtypeStruct((ROWS, VOCAB), jnp.float32),
        compiler_params=pltpu.CompilerParams(
            dimension_semantics=("arbitrary",)
        ),
        interpret=interpret,
    )(tok2)
    return tok2.reshape(ROWS), probs


def kernel(logits, eos_token_ids):
    tokens, probs = _run(logits)
    return tokens, probs
